# baseline (device time: 111129 ns/iter reference)
import jax
import jax.numpy as jnp
from jax import lax
from jax.experimental import pallas as pl
from jax.experimental.pallas import tpu as pltpu

N_DEV = 16
M = 2048
N = 2048
CH = M // N_DEV
NSUB = 4
NS = 2 * NSUB
SB = CH // NSUB
HN = N // 2

MESH = pl.DeviceIdType.MESH

RING = [0, 4, 8, 12, 13, 9, 5, 1, 2, 6, 10, 14, 15, 11, 7, 3]
INV = [0] * N_DEV
for _p, _l in enumerate(RING):
    INV[_l] = _p


def kernel(A, B):
    my = lax.axis_index("i")
    ring = jnp.array(RING, dtype=jnp.int32)
    p = jnp.array(INV, dtype=jnp.int32)[my]
    right = ring[lax.rem(p + 1, N_DEV)]
    left = ring[lax.rem(p - 1 + N_DEV, N_DEV)]
    scalars = jnp.stack([p, left, right]).astype(jnp.int32)

    def silu(z):
        return z * (1.0 / (1.0 + jnp.exp(-z)))

    def body(s_ref, a_ref, b_ref, out_ref, b16_ref,
             sbufs, rbufs, send_sems, recv_sems, credits):
        p = s_ref[0]
        left = s_ref[1]
        right = s_ref[2]

        barrier = pltpu.get_barrier_semaphore()
        for nbr in (left, right):
            pl.semaphore_signal(barrier, inc=1, device_id=(nbr,),
                                device_id_type=MESH)
        pl.semaphore_wait(barrier, 2)

        b16_ref[...] = b_ref[...].astype(jnp.bfloat16)

        def part(idx, col0):
            return jnp.dot(
                a_ref[pl.ds(idx * CH, CH), :].astype(jnp.bfloat16),
                b16_ref[:, pl.ds(col0, HN)],
                preferred_element_type=jnp.float32,
            )

        def stream_dir(s):
            return (right, left) if s % 2 == 0 else (left, right)

        def rows_of(s):
            sub = s // 2
            return slice(sub * SB, (sub + 1) * SB)

        def rs_rdma(s, slot):
            dst, _ = stream_dir(s)
            return pltpu.make_async_remote_copy(
                src_ref=sbufs.at[s, slot], dst_ref=rbufs.at[s, slot],
                send_sem=send_sems.at[s, slot], recv_sem=recv_sems.at[s, slot],
                device_id=(dst,), device_id_type=MESH,
            )

        own_r = part(p, 0)
        own_l = part(p, HN)
        for s in range(NS):
            own = own_r if s % 2 == 0 else own_l
            sbufs[s, 0] = own[rows_of(s)].astype(jnp.bfloat16)
        for s in range(NS):
            rs_rdma(s, 0).start()
        part_r = part(lax.rem(p - 1 + N_DEV, N_DEV), 0)
        part_l = part(lax.rem(p + 1, N_DEV), HN)

        for h in range(N_DEV - 1):
            slot = h % 2
            nslot = (h + 1) % 2
            recv_r = lax.rem(p - h - 1 + 2 * N_DEV, N_DEV)
            recv_l = lax.rem(p + h + 1, N_DEV)
            last = h == N_DEV - 2
            for s in range(NS):
                _, up = stream_dir(s)
                cw = s % 2 == 0
                prt = part_r if cw else part_l
                ridx = recv_r if cw else recv_l
                col0 = 0 if cw else HN
                rows = rows_of(s)
                rs_rdma(s, slot).wait()
                upd = prt[rows] + rbufs[s, slot].astype(jnp.float32)
                if not last:
                    sbufs[s, nslot] = upd.astype(jnp.bfloat16)
                    pl.semaphore_signal(credits.at[s], inc=1, device_id=(up,),
                                        device_id_type=MESH)
                    if h >= 1:
                        pl.semaphore_wait(credits.at[s], 1)
                    rs_rdma(s, nslot).start()
                else:
                    r0 = ridx * CH + (s // 2) * SB
                    out_ref[pl.ds(r0, SB), pl.ds(col0, HN)] = (
                        silu(upd).astype(out_ref.dtype)
                    )
                    pl.semaphore_signal(credits.at[s], inc=1, device_id=(up,),
                                        device_id_type=MESH)
            if not last:
                part_r = part(lax.rem(p - h - 2 + 2 * N_DEV, N_DEV), 0)
                part_l = part(lax.rem(p + h + 2, N_DEV), HN)
        for s in range(NS):
            pl.semaphore_wait(credits.at[s], 2)

        def ag_rdma(s, slot, cidx):
            dst, _ = stream_dir(s)
            col0 = 0 if s % 2 == 0 else HN
            r0 = cidx * CH + (s // 2) * SB
            return pltpu.make_async_remote_copy(
                src_ref=out_ref.at[pl.ds(r0, SB), pl.ds(col0, HN)],
                dst_ref=out_ref.at[pl.ds(r0, SB), pl.ds(col0, HN)],
                send_sem=send_sems.at[s, slot], recv_sem=recv_sems.at[s, slot],
                device_id=(dst,), device_id_type=MESH,
            )

        o_r = lax.rem(p + 1, N_DEV)
        o_l = lax.rem(p - 1 + N_DEV, N_DEV)
        for s in range(NS):
            ag_rdma(s, 0, o_r if s % 2 == 0 else o_l).start()
        for h in range(N_DEV - 1):
            slot = h % 2
            nslot = (h + 1) % 2
            send_r = lax.rem(p - h + 2 * N_DEV, N_DEV)
            send_l = lax.rem(p + h, N_DEV)
            last = h == N_DEV - 2
            for s in range(NS):
                _, up = stream_dir(s)
                cidx = send_r if s % 2 == 0 else send_l
                ag_rdma(s, slot, cidx).wait()
                pl.semaphore_signal(credits.at[s], inc=1, device_id=(up,),
                                    device_id_type=MESH)
                if not last:
                    if h >= 1:
                        pl.semaphore_wait(credits.at[s], 1)
                    ag_rdma(s, nslot, cidx).start()
        for s in range(NS):
            pl.semaphore_wait(credits.at[s], 2)

    return pl.pallas_call(
        body,
        out_shape=jax.ShapeDtypeStruct((M, N), jnp.bfloat16),
        in_specs=[
            pl.BlockSpec(memory_space=pltpu.SMEM),
            pl.BlockSpec(memory_space=pltpu.VMEM),
            pl.BlockSpec(memory_space=pltpu.VMEM),
        ],
        out_specs=pl.BlockSpec(memory_space=pltpu.VMEM),
        scratch_shapes=[
            pltpu.VMEM((1024, N), jnp.bfloat16),
            pltpu.VMEM((NS, 2, SB, HN), jnp.bfloat16),
            pltpu.VMEM((NS, 2, SB, HN), jnp.bfloat16),
            pltpu.SemaphoreType.DMA((NS, 2)),
            pltpu.SemaphoreType.DMA((NS, 2)),
            pltpu.SemaphoreType.REGULAR((NS,)),
        ],
        compiler_params=pltpu.CompilerParams(collective_id=0),
    )(scalars, A, B)


# device time: 108570 ns/iter; 1.0236x vs baseline; 1.0236x over previous
import jax
import jax.numpy as jnp
from jax import lax
from jax.experimental import pallas as pl
from jax.experimental.pallas import tpu as pltpu

N_DEV = 16
M = 2048
N = 2048
CH = M // N_DEV
NSUB = 4
NS = 2 * NSUB
SB = CH // NSUB
HN = N // 2

MESH = pl.DeviceIdType.MESH

RING = [0, 4, 8, 12, 13, 9, 5, 1, 2, 6, 10, 14, 15, 11, 7, 3]
INV = [0] * N_DEV
for _p, _l in enumerate(RING):
    INV[_l] = _p


def kernel(A, B):
    A = A.astype(jnp.bfloat16)
    B = B.astype(jnp.bfloat16)

    my = lax.axis_index("i")
    ring = jnp.array(RING, dtype=jnp.int32)
    p = jnp.array(INV, dtype=jnp.int32)[my]
    right = ring[lax.rem(p + 1, N_DEV)]
    left = ring[lax.rem(p - 1 + N_DEV, N_DEV)]
    scalars = jnp.stack([p, left, right]).astype(jnp.int32)

    def silu(z):
        return z * (1.0 / (1.0 + jnp.exp(-z)))

    def body(s_ref, a_ref, b_ref, out_ref,
             sbufs, rbufs, send_sems, recv_sems, credits):
        p = s_ref[0]
        left = s_ref[1]
        right = s_ref[2]

        barrier = pltpu.get_barrier_semaphore()
        for nbr in (left, right):
            pl.semaphore_signal(barrier, inc=1, device_id=(nbr,),
                                device_id_type=MESH)
        pl.semaphore_wait(barrier, 2)

        def part(idx, col0):
            return jnp.dot(
                a_ref[pl.ds(idx * CH, CH), :],
                b_ref[:, pl.ds(col0, HN)],
                preferred_element_type=jnp.float32,
            )

        def stream_dir(s):
            return (right, left) if s % 2 == 0 else (left, right)

        def rows_of(s):
            sub = s // 2
            return slice(sub * SB, (sub + 1) * SB)

        def rs_rdma(s, slot):
            dst, _ = stream_dir(s)
            return pltpu.make_async_remote_copy(
                src_ref=sbufs.at[s, slot], dst_ref=rbufs.at[s, slot],
                send_sem=send_sems.at[s, slot], recv_sem=recv_sems.at[s, slot],
                device_id=(dst,), device_id_type=MESH,
            )

        own_r = part(p, 0)
        own_l = part(p, HN)
        for s in range(NS):
            own = own_r if s % 2 == 0 else own_l
            sbufs[s, 0] = own[rows_of(s)].astype(jnp.bfloat16)
        for s in range(NS):
            rs_rdma(s, 0).start()
        part_r = part(lax.rem(p - 1 + N_DEV, N_DEV), 0)
        part_l = part(lax.rem(p + 1, N_DEV), HN)

        for h in range(N_DEV - 1):
            slot = h % 2
            nslot = (h + 1) % 2
            recv_r = lax.rem(p - h - 1 + 2 * N_DEV, N_DEV)
            recv_l = lax.rem(p + h + 1, N_DEV)
            last = h == N_DEV - 2
            for s in range(NS):
                _, up = stream_dir(s)
                cw = s % 2 == 0
                prt = part_r if cw else part_l
                ridx = recv_r if cw else recv_l
                col0 = 0 if cw else HN
                rows = rows_of(s)
                rs_rdma(s, slot).wait()
                upd = prt[rows] + rbufs[s, slot].astype(jnp.float32)
                if not last:
                    sbufs[s, nslot] = upd.astype(jnp.bfloat16)
                    pl.semaphore_signal(credits.at[s], inc=1, device_id=(up,),
                                        device_id_type=MESH)
                    if h >= 1:
                        pl.semaphore_wait(credits.at[s], 1)
                    rs_rdma(s, nslot).start()
                else:
                    r0 = ridx * CH + (s // 2) * SB
                    out_ref[pl.ds(r0, SB), pl.ds(col0, HN)] = (
                        silu(upd).astype(out_ref.dtype)
                    )
                    pl.semaphore_signal(credits.at[s], inc=1, device_id=(up,),
                                        device_id_type=MESH)
            if not last:
                part_r = part(lax.rem(p - h - 2 + 2 * N_DEV, N_DEV), 0)
                part_l = part(lax.rem(p + h + 2, N_DEV), HN)
        for s in range(NS):
            pl.semaphore_wait(credits.at[s], 2)

        def ag_rdma(s, slot, cidx):
            dst, _ = stream_dir(s)
            col0 = 0 if s % 2 == 0 else HN
            r0 = cidx * CH + (s // 2) * SB
            return pltpu.make_async_remote_copy(
                src_ref=out_ref.at[pl.ds(r0, SB), pl.ds(col0, HN)],
                dst_ref=out_ref.at[pl.ds(r0, SB), pl.ds(col0, HN)],
                send_sem=send_sems.at[s, slot], recv_sem=recv_sems.at[s, slot],
                device_id=(dst,), device_id_type=MESH,
            )

        o_r = lax.rem(p + 1, N_DEV)
        o_l = lax.rem(p - 1 + N_DEV, N_DEV)
        for s in range(NS):
            ag_rdma(s, 0, o_r if s % 2 == 0 else o_l).start()
        for h in range(N_DEV - 1):
            slot = h % 2
            nslot = (h + 1) % 2
            send_r = lax.rem(p - h + 2 * N_DEV, N_DEV)
            send_l = lax.rem(p + h, N_DEV)
            last = h == N_DEV - 2
            for s in range(NS):
                _, up = stream_dir(s)
                cidx = send_r if s % 2 == 0 else send_l
                ag_rdma(s, slot, cidx).wait()
                pl.semaphore_signal(credits.at[s], inc=1, device_id=(up,),
                                    device_id_type=MESH)
                if not last:
                    if h >= 1:
                        pl.semaphore_wait(credits.at[s], 1)
                    ag_rdma(s, nslot, cidx).start()
        for s in range(NS):
            pl.semaphore_wait(credits.at[s], 2)

    return pl.pallas_call(
        body,
        out_shape=jax.ShapeDtypeStruct((M, N), jnp.bfloat16),
        in_specs=[
            pl.BlockSpec(memory_space=pltpu.SMEM),
            pl.BlockSpec(memory_space=pltpu.VMEM),
            pl.BlockSpec(memory_space=pltpu.VMEM),
        ],
        out_specs=pl.BlockSpec(memory_space=pltpu.VMEM),
        scratch_shapes=[
            pltpu.VMEM((NS, 2, SB, HN), jnp.bfloat16),
            pltpu.VMEM((NS, 2, SB, HN), jnp.bfloat16),
            pltpu.SemaphoreType.DMA((NS, 2)),
            pltpu.SemaphoreType.DMA((NS, 2)),
            pltpu.SemaphoreType.REGULAR((NS,)),
        ],
        compiler_params=pltpu.CompilerParams(collective_id=0),
    )(scalars, A, B)
